# baseline (speedup 1.0000x reference)
import functools

import jax
import jax.numpy as jnp
from jax import lax
from jax.experimental import pallas as pl
from jax.experimental.pallas import tpu as pltpu
from jax.experimental.pallas import tpu_sc as plsc

B, L, D = 4096, 200, 64
MAX_POS = 252
N = B * L
NC, NS = 2, 16
NW = NC * NS
PER_W = N // NW
CHUNK = 512
SUB = 128
NSUB = CHUNK // SUB
NCHUNK = PER_W // CHUNK
NPAIR = NCHUNK // 2

_mesh = plsc.VectorSubcoreMesh(core_axis_name="c", subcore_axis_name="s")


@functools.partial(
    pl.kernel,
    mesh=_mesh,
    out_type=jax.ShapeDtypeStruct((N, D), jnp.float32),
    compiler_params=pltpu.CompilerParams(use_tc_tiling_on_sc=False,
                                         needs_layout_passes=False),
    scratch_types=[
        pltpu.VMEM((2, CHUNK), jnp.int32),
        pltpu.VMEM((2, CHUNK), jnp.int32),
        pltpu.VMEM((2, CHUNK, D), jnp.float32),
        pltpu.VMEM((MAX_POS, D), jnp.float32),
        pltpu.SemaphoreType.DMA,
        pltpu.SemaphoreType.DMA,
        pltpu.SemaphoreType.DMA,
        pltpu.SemaphoreType.DMA,
        pltpu.SemaphoreType.DMA,
        pltpu.SemaphoreType.DMA,
    ],
)
def _embed_kernel(x_hbm, seq_hbm, tab_hbm, pos_hbm, out_hbm,
                  tok_idx, pos_idx, tok_rows, pos_local,
                  sem_in0, sem_in1, sem_out0, sem_out1, sem_idx0, sem_idx1):
    sem_in = (sem_in0, sem_in1)
    sem_out = (sem_out0, sem_out1)
    sem_idx = (sem_idx0, sem_idx1)
    wid = lax.axis_index("s") * NC + lax.axis_index("c")
    base0 = wid * PER_W
    lane = lax.iota(jnp.int32, 16)

    def issue_gathers(b):
        for j in range(NSUB):
            s = pl.ds(j * SUB, SUB)
            pltpu.async_copy(tab_hbm.at[tok_idx.at[b, s]],
                             tok_rows.at[b, s], sem_in[b])

    def drain_gathers(b):
        pltpu.make_async_copy(tab_hbm.at[pl.ds(0, CHUNK)],
                              tok_rows.at[b], sem_in[b]).wait()

    def issue_idx(gdyn, b):
        base = base0 + gdyn * CHUNK
        pltpu.async_copy(x_hbm.at[pl.ds(base, CHUNK)], tok_idx.at[b], sem_idx[b])
        pltpu.async_copy(seq_hbm.at[pl.ds(base, CHUNK)], pos_idx.at[b], sem_idx[b])

    def drain_idx(b):
        pltpu.make_async_copy(x_hbm.at[pl.ds(0, CHUNK)],
                              tok_idx.at[b], sem_idx[b]).wait()
        pltpu.make_async_copy(x_hbm.at[pl.ds(0, CHUNK)],
                              pos_idx.at[b], sem_idx[b]).wait()

    def issue_out(gdyn, b):
        base = base0 + gdyn * CHUNK
        pltpu.async_copy(tok_rows.at[b], out_hbm.at[pl.ds(base, CHUNK)], sem_out[b])

    def drain_out(b):
        pltpu.make_async_copy(tok_rows.at[b], out_hbm.at[pl.ds(0, CHUNK)],
                              sem_out[b]).wait()

    def add_chunk(b):
        rows2d = tok_rows.at[b]

        def add_block(r, carry):
            pvec = pos_idx[b, pl.ds(r * 16, 16)]
            for j in range(D):
                jvec = jnp.full((16,), j, dtype=jnp.int32)
                pos_v = plsc.load_gather(pos_local, [pvec, jvec])
                plsc.addupdate_scatter(rows2d.at[pl.ds(r * 16, 16)],
                                       [lane, jvec], pos_v)
            return carry

        lax.fori_loop(0, CHUNK // 16, add_block, 0)

    pltpu.sync_copy(pos_hbm, pos_local)
    pltpu.sync_copy(x_hbm.at[pl.ds(base0, CHUNK)], tok_idx.at[0])
    pltpu.sync_copy(seq_hbm.at[pl.ds(base0, CHUNK)], pos_idx.at[0])
    issue_gathers(0)
    issue_idx(1, 1)

    def pair_body(g2, carry):
        for b in range(2):
            g = 2 * g2 + b
            nb = 1 - b
            if b == 0:
                drain_idx(nb)

                @pl.when(g2 >= 1)
                def _():
                    drain_out(nb)
                issue_gathers(nb)
            else:
                @pl.when(g2 < NPAIR - 1)
                def _():
                    drain_idx(nb)
                    drain_out(nb)
                    issue_gathers(nb)
            drain_gathers(b)
            add_chunk(b)
            @pl.when(g2 < NPAIR - 1)
            def _():
                issue_idx(g + 2, b)
            issue_out(g, b)
        return carry

    lax.fori_loop(0, NPAIR, pair_body, 0)
    drain_out(0)
    drain_out(1)


def kernel(x, seq_idx, embed_table, pos_table):
    x_flat = x.reshape(-1).astype(jnp.int32)
    seq_flat = seq_idx.reshape(-1).astype(jnp.int32)
    out = _embed_kernel(x_flat, seq_flat, embed_table, pos_table)
    return out.reshape(B, L, D)


# speedup vs baseline: 1.8712x; 1.8712x over previous
import functools

import jax
import jax.numpy as jnp
from jax import lax
from jax.experimental import pallas as pl
from jax.experimental.pallas import tpu as pltpu
from jax.experimental.pallas import tpu_sc as plsc

B, L, D = 4096, 200, 64
MAX_POS = 252
N = B * L
NC, NS = 2, 16
NW = NC * NS
PER_W = N // NW
CHUNK = 512
SUB = 128
NSUB = CHUNK // SUB
NCHUNK = PER_W // CHUNK
NPAIR = NCHUNK // 2

_mesh = plsc.VectorSubcoreMesh(core_axis_name="c", subcore_axis_name="s")


@functools.partial(
    pl.kernel,
    mesh=_mesh,
    out_type=jax.ShapeDtypeStruct((N, D), jnp.float32),
    compiler_params=pltpu.CompilerParams(use_tc_tiling_on_sc=False,
                                         needs_layout_passes=False),
    scratch_types=[
        pltpu.VMEM((2, CHUNK), jnp.int32),
        pltpu.VMEM((2, CHUNK), jnp.int32),
        pltpu.VMEM((2, CHUNK, D), jnp.float32),
        pltpu.VMEM((MAX_POS, D), jnp.float32),
        pltpu.SemaphoreType.DMA,
        pltpu.SemaphoreType.DMA,
        pltpu.SemaphoreType.DMA,
        pltpu.SemaphoreType.DMA,
        pltpu.SemaphoreType.DMA,
        pltpu.SemaphoreType.DMA,
    ],
)
def _embed_kernel(x_hbm, seq_hbm, tab_hbm, pos_hbm, out_hbm,
                  tok_idx, pos_idx, tok_rows, pos_local,
                  sem_in0, sem_in1, sem_out0, sem_out1, sem_idx0, sem_idx1):
    sem_in = (sem_in0, sem_in1)
    sem_out = (sem_out0, sem_out1)
    sem_idx = (sem_idx0, sem_idx1)
    wid = lax.axis_index("s") * NC + lax.axis_index("c")
    base0 = wid * PER_W
    lane = lax.iota(jnp.int32, 16)

    def issue_gathers(b):
        for j in range(NSUB):
            s = pl.ds(j * SUB, SUB)
            pltpu.async_copy(tab_hbm.at[tok_idx.at[b, s]],
                             tok_rows.at[b, s], sem_in[b])

    def drain_gathers(b):
        pltpu.make_async_copy(tab_hbm.at[pl.ds(0, CHUNK)],
                              tok_rows.at[b], sem_in[b]).wait()

    def issue_idx(gdyn, b):
        base = base0 + gdyn * CHUNK
        pltpu.async_copy(x_hbm.at[pl.ds(base, CHUNK)], tok_idx.at[b], sem_idx[b])
        pltpu.async_copy(seq_hbm.at[pl.ds(base, CHUNK)], pos_idx.at[b], sem_idx[b])

    def drain_idx(b):
        pltpu.make_async_copy(x_hbm.at[pl.ds(0, CHUNK)],
                              tok_idx.at[b], sem_idx[b]).wait()
        pltpu.make_async_copy(x_hbm.at[pl.ds(0, CHUNK)],
                              pos_idx.at[b], sem_idx[b]).wait()

    def issue_out(gdyn, b):
        base = base0 + gdyn * CHUNK
        pltpu.async_copy(tok_rows.at[b], out_hbm.at[pl.ds(base, CHUNK)], sem_out[b])

    def drain_out(b):
        pltpu.make_async_copy(tok_rows.at[b], out_hbm.at[pl.ds(0, CHUNK)],
                              sem_out[b]).wait()

    def add_chunk(b):
        rows2d = tok_rows.at[b]

        def add_block(r, carry):
            pvec = pos_idx[b, pl.ds(r * 16, 16)]
            block = rows2d.at[pl.ds(r * 16, 16)]
            for t in range(16):
                rot = (lane + t) & 15
                for cb in range(0, D, 16):
                    jvec = rot + cb
                    pos_v = plsc.load_gather(pos_local, [pvec, jvec])
                    plsc.addupdate_scatter(block, [lane, jvec], pos_v)
            return carry

        lax.fori_loop(0, CHUNK // 16, add_block, 0)

    pltpu.sync_copy(pos_hbm, pos_local)
    pltpu.sync_copy(x_hbm.at[pl.ds(base0, CHUNK)], tok_idx.at[0])
    pltpu.sync_copy(seq_hbm.at[pl.ds(base0, CHUNK)], pos_idx.at[0])
    issue_gathers(0)
    issue_idx(1, 1)

    def pair_body(g2, carry):
        for b in range(2):
            g = 2 * g2 + b
            nb = 1 - b
            if b == 0:
                drain_idx(nb)

                @pl.when(g2 >= 1)
                def _():
                    drain_out(nb)
                issue_gathers(nb)
            else:
                @pl.when(g2 < NPAIR - 1)
                def _():
                    drain_idx(nb)
                    drain_out(nb)
                    issue_gathers(nb)
            drain_gathers(b)
            add_chunk(b)
            @pl.when(g2 < NPAIR - 1)
            def _():
                issue_idx(g + 2, b)
            issue_out(g, b)
        return carry

    lax.fori_loop(0, NPAIR, pair_body, 0)
    drain_out(0)
    drain_out(1)


def kernel(x, seq_idx, embed_table, pos_table):
    x_flat = x.reshape(-1).astype(jnp.int32)
    seq_flat = seq_idx.reshape(-1).astype(jnp.int32)
    out = _embed_kernel(x_flat, seq_flat, embed_table, pos_table)
    return out.reshape(B, L, D)


# speedup vs baseline: 1.8762x; 1.0027x over previous
import functools

import jax
import jax.numpy as jnp
from jax import lax
from jax.experimental import pallas as pl
from jax.experimental.pallas import tpu as pltpu
from jax.experimental.pallas import tpu_sc as plsc

B, L, D = 4096, 200, 64
MAX_POS = 252
NC, NS = 2, 16
NW = NC * NS
B_PER_W = B // NW
NB = 2
ROWS = NB * L
NCHUNK = B_PER_W // NB
NPAIR = NCHUNK // 2

_mesh = plsc.VectorSubcoreMesh(core_axis_name="c", subcore_axis_name="s")


@functools.partial(
    pl.kernel,
    mesh=_mesh,
    out_type=jax.ShapeDtypeStruct((B, L, D), jnp.float32),
    compiler_params=pltpu.CompilerParams(use_tc_tiling_on_sc=False,
                                         needs_layout_passes=False),
    scratch_types=[
        pltpu.VMEM((2, ROWS), jnp.int32),
        pltpu.VMEM((2, ROWS), jnp.int32),
        pltpu.VMEM((2, ROWS, D), jnp.float32),
        pltpu.VMEM((MAX_POS, D), jnp.float32),
        pltpu.SemaphoreType.DMA,
        pltpu.SemaphoreType.DMA,
        pltpu.SemaphoreType.DMA,
        pltpu.SemaphoreType.DMA,
        pltpu.SemaphoreType.DMA,
        pltpu.SemaphoreType.DMA,
    ],
)
def _embed_kernel(x_hbm, seq_hbm, tab_hbm, pos_hbm, out_hbm,
                  tok_idx, pos_idx, tok_rows, pos_local,
                  sem_in0, sem_in1, sem_out0, sem_out1, sem_idx0, sem_idx1):
    sem_in = (sem_in0, sem_in1)
    sem_out = (sem_out0, sem_out1)
    sem_idx = (sem_idx0, sem_idx1)
    wid = lax.axis_index("s") * NC + lax.axis_index("c")
    b_base = wid * B_PER_W
    lane = lax.iota(jnp.int32, 16)
    SPLITS = ((0, 128), (128, 72))

    def issue_gathers(b):
        for k in range(NB):
            for off, ln in SPLITS:
                s = pl.ds(k * L + off, ln)
                pltpu.async_copy(tab_hbm.at[tok_idx.at[b, s]],
                                 tok_rows.at[b, s], sem_in[b])

    def drain_gathers(b):
        pltpu.make_async_copy(tab_hbm.at[pl.ds(0, ROWS)],
                              tok_rows.at[b], sem_in[b]).wait()

    def issue_idx(gdyn, b):
        b0 = b_base + gdyn * NB
        for k in range(NB):
            s = pl.ds(k * L, L)
            pltpu.async_copy(x_hbm.at[b0 + k], tok_idx.at[b, s], sem_idx[b])
            pltpu.async_copy(seq_hbm.at[b0 + k], pos_idx.at[b, s], sem_idx[b])

    def drain_idx(b):
        pltpu.make_async_copy(x_hbm.at[0], tok_idx.at[b, pl.ds(0, L)],
                              sem_idx[b]).wait()
        pltpu.make_async_copy(x_hbm.at[0], tok_idx.at[b, pl.ds(L, L)],
                              sem_idx[b]).wait()
        pltpu.make_async_copy(x_hbm.at[0], pos_idx.at[b, pl.ds(0, L)],
                              sem_idx[b]).wait()
        pltpu.make_async_copy(x_hbm.at[0], pos_idx.at[b, pl.ds(L, L)],
                              sem_idx[b]).wait()

    def issue_out(gdyn, b):
        b0 = b_base + gdyn * NB
        for k in range(NB):
            pltpu.async_copy(tok_rows.at[b, pl.ds(k * L, L)],
                             out_hbm.at[b0 + k], sem_out[b])

    def drain_out(b):
        for k in range(NB):
            pltpu.make_async_copy(tok_rows.at[b, pl.ds(k * L, L)],
                                  out_hbm.at[0], sem_out[b]).wait()

    def add_chunk(b):
        rows2d = tok_rows.at[b]

        def add_block(r, carry):
            pvec = pos_idx[b, pl.ds(r * 16, 16)]
            block = rows2d.at[pl.ds(r * 16, 16)]
            for t in range(16):
                rot = (lane + t) & 15
                for cb in range(0, D, 16):
                    jvec = rot + cb
                    pos_v = plsc.load_gather(pos_local, [pvec, jvec])
                    plsc.addupdate_scatter(block, [lane, jvec], pos_v)
            return carry

        lax.fori_loop(0, ROWS // 16, add_block, 0)

    pltpu.sync_copy(pos_hbm, pos_local)
    for k in range(NB):
        s = pl.ds(k * L, L)
        pltpu.sync_copy(x_hbm.at[b_base + k], tok_idx.at[0, s])
        pltpu.sync_copy(seq_hbm.at[b_base + k], pos_idx.at[0, s])
    issue_gathers(0)
    issue_idx(1, 1)

    def pair_body(g2, carry):
        for b in range(2):
            g = 2 * g2 + b
            nb = 1 - b
            if b == 0:
                drain_idx(nb)

                @pl.when(g2 >= 1)
                def _():
                    drain_out(nb)
                issue_gathers(nb)
            else:
                @pl.when(g2 < NPAIR - 1)
                def _():
                    drain_idx(nb)
                    drain_out(nb)
                    issue_gathers(nb)
            drain_gathers(b)
            add_chunk(b)
            @pl.when(g2 < NPAIR - 1)
            def _():
                issue_idx(g + 2, b)
            issue_out(g, b)
        return carry

    lax.fori_loop(0, NPAIR, pair_body, 0)
    drain_out(0)
    drain_out(1)


def kernel(x, seq_idx, embed_table, pos_table):
    return _embed_kernel(x.astype(jnp.int32), seq_idx.astype(jnp.int32),
                         embed_table, pos_table)
